# baseline (speedup 1.0000x reference)
import functools

import jax
import jax.numpy as jnp
from jax import lax
from jax.experimental import pallas as pl
from jax.experimental.pallas import tpu as pltpu

N = 8192
D = 256
BR = 512
NB = N // BR
BIG_I = 2**30

_DN = (((1,), (1,)), ((), ()))


def _nn_body(u_ref, loc_ref, val_ref, out_ref, best_d, best_i, row_v, sem):
    i = pl.program_id(0)

    d = loc_ref[...] - u_ref[...]
    d2 = d * d
    ones = jnp.ones((8, D), jnp.float32)
    s = lax.dot_general(ones, d2, _DN, preferred_element_type=jnp.float32)
    gi = i * BR + lax.broadcasted_iota(jnp.int32, (8, BR), 1)

    @pl.when(i == 0)
    def _init():
        best_d[...] = s
        best_i[...] = gi

    @pl.when(i > 0)
    def _update():
        mask = s < best_d[...]
        best_d[...] = jnp.where(mask, s, best_d[...])
        best_i[...] = jnp.where(mask, gi, best_i[...])

    @pl.when(i == NB - 1)
    def _gather():
        m = jnp.min(best_d[...])
        idx = jnp.min(jnp.where(best_d[...] == m, best_i[...], BIG_I))
        copy = pltpu.make_async_copy(val_ref.at[idx], row_v, sem)
        copy.start()
        copy.wait()
        out_ref[...] = row_v[...]


@functools.partial(jax.jit, static_argnames=())
def _nn(u2, loc, val):
    return pl.pallas_call(
        _nn_body,
        grid=(NB,),
        in_specs=[
            pl.BlockSpec((1, D), lambda i: (0, 0)),
            pl.BlockSpec((BR, D), lambda i: (i, 0)),
            pl.BlockSpec(memory_space=pl.ANY),
        ],
        out_specs=pl.BlockSpec(memory_space=pltpu.VMEM),
        out_shape=jax.ShapeDtypeStruct((D,), jnp.float32),
        scratch_shapes=[
            pltpu.VMEM((8, BR), jnp.float32),
            pltpu.VMEM((8, BR), jnp.int32),
            pltpu.VMEM((D,), jnp.float32),
            pltpu.SemaphoreType.DMA,
        ],
    )(u2, loc, val)


def kernel(u, loc, val, p):
    del p
    return _nn(u.reshape(1, D), loc, val)


# speedup vs baseline: 1.9586x; 1.9586x over previous
import functools

import jax
import jax.numpy as jnp
from jax import lax
from jax.experimental import pallas as pl
from jax.experimental.pallas import tpu as pltpu

N = 8192
D = 256
BR = 2048
NB = N // BR
BIG_I = 2**30

_DN = (((1,), (1,)), ((), ()))


def _nn_body(u_ref, loc_ref, val_ref, out_ref, best_d, best_i, row_v, sem):
    i = pl.program_id(0)

    d = loc_ref[...] - u_ref[...]
    d2 = d * d
    ones = jnp.ones((8, D), jnp.float32)
    s = lax.dot_general(ones, d2, _DN, preferred_element_type=jnp.float32)
    gi = i * BR + lax.broadcasted_iota(jnp.int32, (8, BR), 1)

    @pl.when(i == 0)
    def _init():
        best_d[...] = s
        best_i[...] = gi

    @pl.when(i > 0)
    def _update():
        mask = s < best_d[...]
        best_d[...] = jnp.where(mask, s, best_d[...])
        best_i[...] = jnp.where(mask, gi, best_i[...])

    @pl.when(i == NB - 1)
    def _gather():
        m = jnp.min(best_d[...])
        idx = jnp.min(jnp.where(best_d[...] == m, best_i[...], BIG_I))
        copy = pltpu.make_async_copy(val_ref.at[idx], row_v, sem)
        copy.start()
        copy.wait()
        out_ref[...] = row_v[...]


@functools.partial(jax.jit, static_argnames=())
def _nn(u2, loc, val):
    return pl.pallas_call(
        _nn_body,
        grid=(NB,),
        in_specs=[
            pl.BlockSpec((1, D), lambda i: (0, 0)),
            pl.BlockSpec((BR, D), lambda i: (i, 0)),
            pl.BlockSpec(memory_space=pl.ANY),
        ],
        out_specs=pl.BlockSpec(memory_space=pltpu.VMEM),
        out_shape=jax.ShapeDtypeStruct((D,), jnp.float32),
        scratch_shapes=[
            pltpu.VMEM((8, BR), jnp.float32),
            pltpu.VMEM((8, BR), jnp.int32),
            pltpu.VMEM((D,), jnp.float32),
            pltpu.SemaphoreType.DMA,
        ],
    )(u2, loc, val)


def kernel(u, loc, val, p):
    del p
    return _nn(u.reshape(1, D), loc, val)
